# Initial kernel scaffold; baseline (speedup 1.0000x reference)
import jax
import jax.numpy as jnp
from jax.experimental import pallas as pl


def kernel(inputs, roi_label, detectionscore, lut, cq):
    raise NotImplementedError("write your pallas kernel here")


# speedup vs baseline: 3.5788x; 3.5788x over previous
import functools

import jax
import jax.numpy as jnp
from jax.experimental import pallas as pl
from jax.experimental.pallas import tpu as pltpu

N = 4096
F = 256
L = 5532
Q = 5000
SCALAR = 30.0
BC = 512
NLB = (L + BC - 1) // BC
NQB = (Q + BC - 1) // BC
NB = NLB + NQB
NEG = -1e30


def _oim_body(x_ref, lut_ref, cq_ref, lbl_ref, out_ref, m_s, s_s, g_s):
    j = pl.program_id(0)
    is_lut = j < NLB

    @pl.when(j == 0)
    def _init():
        m_s[...] = jnp.full((N, 1), NEG, dtype=jnp.float32)
        s_s[...] = jnp.zeros((N, 1), dtype=jnp.float32)
        g_s[...] = jnp.zeros((N, 1), dtype=jnp.float32)

    t = jnp.where(is_lut, lut_ref[...], cq_ref[...])
    x = x_ref[...]
    logits = SCALAR * jax.lax.dot_general(
        x, t, (((1,), (1,)), ((), ())),
        preferred_element_type=jnp.float32)

    base = jnp.where(is_lut, j * BC, L + (j - NLB) * BC)
    limit = jnp.where(is_lut, L, L + Q)
    col = base + jax.lax.broadcasted_iota(jnp.int32, (1, BC), 1)
    masked = jnp.where(col < limit, logits, NEG)

    lbl = lbl_ref[...].astype(jnp.int32)
    hit = col == lbl
    g_s[...] += jnp.sum(jnp.where(hit, masked, 0.0), axis=1, keepdims=True)

    m_old = m_s[...]
    m_new = jnp.maximum(m_old, jnp.max(masked, axis=1, keepdims=True))
    s_s[...] = (s_s[...] * jnp.exp(m_old - m_new)
                + jnp.sum(jnp.exp(masked - m_new), axis=1, keepdims=True))
    m_s[...] = m_new

    @pl.when(j == NB - 1)
    def _finish():
        lblf = lbl_ref[...]
        valid = lblf >= 0.0
        nll = m_s[...] + jnp.log(s_s[...]) - g_s[...]
        loss_sum = jnp.sum(jnp.where(valid, nll, 0.0), keepdims=True)
        cnt = jnp.sum(valid.astype(jnp.float32), keepdims=True)
        out_ref[...] = loss_sum / jnp.maximum(cnt, 1.0)


@jax.jit
def _oim_loss(inputs, label_f, lut, cq):
    out = pl.pallas_call(
        _oim_body,
        grid=(NB,),
        in_specs=[
            pl.BlockSpec((N, F), lambda j: (0, 0)),
            pl.BlockSpec((BC, F), lambda j: (jnp.minimum(j, NLB - 1), 0)),
            pl.BlockSpec((BC, F), lambda j: (jnp.maximum(j - NLB, 0), 0)),
            pl.BlockSpec((N, 1), lambda j: (0, 0)),
        ],
        out_specs=pl.BlockSpec((1, 1), lambda j: (0, 0)),
        out_shape=jax.ShapeDtypeStruct((1, 1), jnp.float32),
        scratch_shapes=[
            pltpu.VMEM((N, 1), jnp.float32),
            pltpu.VMEM((N, 1), jnp.float32),
            pltpu.VMEM((N, 1), jnp.float32),
        ],
        compiler_params=pltpu.CompilerParams(
            dimension_semantics=("arbitrary",)),
    )(inputs, lut, cq, label_f)
    return out[0, 0]


def kernel(inputs, roi_label, detectionscore, lut, cq):
    label_f = (roi_label.reshape(-1, 1) - 1).astype(jnp.float32)
    loss = _oim_loss(inputs, label_f, lut, cq)
    return (loss, lut)
